# baseline (speedup 1.0000x reference)
import functools

import jax
import jax.numpy as jnp
from jax import lax
from jax.experimental import pallas as pl
from jax.experimental.pallas import tpu as pltpu
from jax.experimental.pallas import tpu_sc as plsc

K = 5
V = 512
C = 16
NG = 16
NEB = 8192
NODES = NG * V
NK = K * K
EPT = NEB // 16
GPS = NG // 2


_YR = NODES * C // 128


def _prep_body(xg8_ref, w8_ref, y_ref):
    y_ref[...] = lax.dot_general(xg8_ref[...], w8_ref[0],
                                 (((1,), (0,)), ((), ())),
                                 preferred_element_type=jnp.float32)


_prep = pl.pallas_call(
    _prep_body,
    grid=(NK,),
    in_specs=[
        pl.BlockSpec((_YR, 128), lambda k: (0, 0)),
        pl.BlockSpec((1, 128, 128), lambda k: (k, 0, 0)),
    ],
    out_specs=pl.BlockSpec((_YR, 128), lambda k: (k, 0)),
    out_shape=jax.ShapeDtypeStruct((NK * _YR, 128), jnp.float32),
)


def _tail_body(xg_ref, wrr_ref, b2_ref, bres2_ref, tail_ref):
    p = lax.dot_general(xg_ref[...], wrr_ref[...], (((1,), (0,)), ((), ())),
                        preferred_element_type=jnp.float32)
    root = p[:, :C] + b2_ref[...]
    r = p[:, C:] + bres2_ref[...]
    res = jnp.where(r > 0, r, jnp.exp(r) - 1.0)
    zer = jnp.zeros(root.shape[:1] + (6 * C,), jnp.float32)
    tail_ref[...] = jnp.concatenate([root, res, zer], axis=1)


_tail = pl.pallas_call(
    _tail_body,
    out_shape=jax.ShapeDtypeStruct((NODES, 128), jnp.float32),
)


def _lane_bcast(vec, lane):
    return lax.gather(
        vec, jnp.full((16, 1), lane, jnp.int32),
        lax.GatherDimensionNumbers(offset_dims=(), collapsed_slice_dims=(0,),
                                   start_index_map=(0,)),
        (1,), mode=lax.GatherScatterMode.PROMISE_IN_BOUNDS)


def _edges_body(y_hbm, ea0_hbm, ea1_hbm, src_hbm, dst_hbm, tail_hbm,
                out_hbm,
                ia0, ia1, ia2, ia3, ib0, ib1, ib2, ib3, dst_v,
                bas0, bas1, bas2, bas3, ea0_v, ea1_v, src_v,
                ra0, ra1, ra2, ra3, rb0, rb1, rb2, rb3,
                m_v, agg_sp, cnt_sp, sem_a, sem_b):
    c = lax.axis_index("c")
    s = lax.axis_index("s")
    e0 = s * EPT
    rpt = GPS * V // 16

    idx_a = (ia0, ia1, ia2, ia3)
    idx_b = (ib0, ib1, ib2, ib3)
    rows_a = (ra0, ra1, ra2, ra3)
    rows_b = (rb0, rb1, rb2, rb3)

    @plsc.parallel_loop(0, rpt, step=1, unroll=4)
    def _fill(j):
        m_v[j] = jnp.zeros((C,), jnp.float32)
        ra0[j] = jnp.ones((C,), jnp.float32)
        ra0[j + rpt] = jnp.ones((C,), jnp.float32)

    pltpu.sync_copy(m_v.at[pl.ds(0, rpt)], agg_sp.at[pl.ds(s * rpt, rpt)])
    pltpu.sync_copy(m_v.at[pl.ds(0, V // 16)],
                    cnt_sp.at[pl.ds(s * (V // 16), V // 16)])

    pltpu.sync_copy(ea0_hbm.at[pl.ds(e0, EPT)], ea0_v)
    pltpu.sync_copy(ea1_hbm.at[pl.ds(e0, EPT)], ea1_v)
    pltpu.sync_copy(src_hbm.at[pl.ds(e0, EPT)], src_v)
    pltpu.sync_copy(dst_hbm.at[pl.ds(e0, EPT)], dst_v)

    off0 = c * (GPS * V)

    @plsc.parallel_loop(0, EPT // 16, step=1, unroll=2)
    def _setup(j):
        sl = pl.ds(j * 16, 16)
        p0 = ea0_v[sl] * (K - 1.0)
        p1 = ea1_v[sl] * (K - 1.0)
        i0 = jnp.minimum(p0.astype(jnp.int32), K - 2)
        i1 = jnp.minimum(p1.astype(jnp.int32), K - 2)
        f0 = p0 - i0.astype(jnp.float32)
        f1 = p1 - i1.astype(jnp.float32)
        g0 = 1.0 - f0
        g1 = 1.0 - f1
        bas0[sl] = g0 * g1
        bas1[sl] = f0 * g1
        bas2[sl] = g0 * f1
        bas3[sl] = f0 * f1
        wib = src_v[sl] + (i0 + i1 * K) * NODES + off0
        ia0[sl] = wib
        ia1[sl] = wib + NODES
        ia2[sl] = wib + K * NODES
        ia3[sl] = wib + (K + 1) * NODES

    def _addv(dref, sref, nchunks, val):
        @plsc.parallel_loop(0, nchunks, step=1, unroll=2)
        def f(j):
            dref[pl.ds(j * 16, 16)] = sref[pl.ds(j * 16, 16)] + val

    plsc.subcore_barrier()

    pltpu.sync_copy(ra0, cnt_sp.at[dst_v], add=True)

    bufs = ((idx_a, rows_a, sem_a), (idx_b, rows_b, sem_b))
    descs = [pltpu.async_copy(y_hbm.at[ix], r, sem_a)
             for ix, r in zip(idx_a, rows_a)]
    for g in range(GPS):
        cur_i, cur_r, _ = bufs[g % 2]
        nxt_i, nxt_r, nxt_s = bufs[(g + 1) % 2]
        if g < GPS - 1:
            with jax.named_scope("idxprep"):
                for corner in range(4):
                    _addv(nxt_i[corner], cur_i[corner], EPT // 16, V)
        with jax.named_scope("dwait"):
            for d in descs:
                d.wait()
        if g < GPS - 1:
            descs = [pltpu.async_copy(y_hbm.at[ix], r, nxt_s)
                     for ix, r in zip(nxt_i, nxt_r)]

        c0, c1, c2, c3 = cur_r

        with jax.named_scope("fma"):
            @plsc.parallel_loop(0, EPT // 16, step=1, unroll=1)
            def _body(ch):
                base = ch * 16
                sl = pl.ds(base, 16)
                b0 = bas0[sl]
                b1 = bas1[sl]
                b2v = bas2[sl]
                b3 = bas3[sl]
                for u in range(16):
                    ee = base + u
                    m = (c0[ee] * _lane_bcast(b0, u)
                         + c1[ee] * _lane_bcast(b1, u)
                         + c2[ee] * _lane_bcast(b2v, u)
                         + c3[ee] * _lane_bcast(b3, u))
                    m_v[ee] = m

        with jax.named_scope("scat"):
            pltpu.sync_copy(m_v, agg_sp.at[dst_v], add=True)
        if g < GPS - 1:
            with jax.named_scope("dstinc"):
                _addv(dst_v, dst_v, EPT // 16, V)

    plsc.subcore_barrier()

    row0 = s * rpt
    gbase = c * (GPS * V) + row0
    v_off = lax.rem(s, 2) * rpt
    pltpu.sync_copy(agg_sp.at[pl.ds(row0, rpt)], m_v.at[pl.ds(0, rpt)])
    pltpu.sync_copy(cnt_sp.at[pl.ds(v_off, rpt)], ra2.at[pl.ds(0, rpt)])
    pltpu.sync_copy(tail_hbm.at[pl.ds(gbase, rpt), pl.ds(0, C)],
                    ra0.at[pl.ds(0, rpt)])
    pltpu.sync_copy(tail_hbm.at[pl.ds(gbase, rpt), pl.ds(C, C)],
                    ra1.at[pl.ds(0, rpt)])

    def _fin(j, carry):
        cntv = jnp.maximum(ra2[j], 1.0)
        xo = m_v[j] / cntv + ra0[j]
        xo = jnp.where(xo > 0, xo, jnp.exp(xo) - 1.0)
        xo = xo + ra1[j]
        xo = jnp.where(xo > 0, xo, jnp.exp(xo) - 1.0)
        m_v[j + rpt] = xo
        return carry
    lax.fori_loop(0, rpt, _fin, 0)
    pltpu.sync_copy(m_v.at[pl.ds(rpt, rpt)], out_hbm.at[pl.ds(gbase, rpt)])


@functools.cache
def _get_edges():
    mesh = plsc.VectorSubcoreMesh(core_axis_name="c", subcore_axis_name="s",
                                  num_cores=2, num_subcores=16)
    idx_t = pltpu.VMEM((EPT,), jnp.int32)
    vec_t = pltpu.VMEM((EPT,), jnp.float32)
    row_t = pltpu.VMEM((EPT, C), jnp.float32)
    return pl.kernel(
        _edges_body,
        mesh=mesh,
        compiler_params=pltpu.CompilerParams(use_tc_tiling_on_sc=False),
        out_type=jax.ShapeDtypeStruct((NODES, C), jnp.float32),
        scratch_types=[
            idx_t, idx_t, idx_t, idx_t,
            idx_t, idx_t, idx_t, idx_t,
            idx_t,
            vec_t, vec_t, vec_t, vec_t,
            vec_t, vec_t,
            idx_t,
            row_t, row_t, row_t, row_t,
            row_t, row_t, row_t, row_t,
            row_t,
            pltpu.VMEM_SHARED((GPS * V, C), jnp.float32),
            pltpu.VMEM_SHARED((V, C), jnp.float32),
            pltpu.SemaphoreType.DMA,
            pltpu.SemaphoreType.DMA,
        ],
    )


def kernel(x, edge_index, edge_attr, Wspline, Wroot, b, Wres, bres):
    n, v, cc, t = x.shape
    xg = x.transpose(3, 0, 1, 2).reshape(NODES, C)

    wrr = jnp.concatenate([Wroot, Wres.T], axis=1)
    b2 = b.reshape(1, C)
    bres2 = bres.reshape(1, C)

    ea0 = edge_attr[:NEB, 0]
    ea1 = edge_attr[:NEB, 1]
    src = edge_index[0, :NEB]
    dst = edge_index[1, :NEB]

    w8 = jax.vmap(lambda w: jnp.kron(jnp.eye(8, dtype=w.dtype), w))(Wspline)
    xg128 = xg.reshape(_YR, 128)

    y = _prep(xg128, w8)
    tail = _tail(xg, wrr, b2, bres2)
    yflat = y.reshape(NK * NODES, C)

    out_node = _get_edges()(yflat, ea0, ea1, src, dst, tail)

    return out_node.reshape(t, n, v, C).transpose(1, 2, 3, 0)


# speedup vs baseline: 1.0746x; 1.0746x over previous
import functools

import jax
import jax.numpy as jnp
from jax import lax
from jax.experimental import pallas as pl
from jax.experimental.pallas import tpu as pltpu
from jax.experimental.pallas import tpu_sc as plsc

K = 5
V = 512
C = 16
NG = 16
NEB = 8192
NODES = NG * V
NK = K * K
EPT = NEB // 16
GPS = NG // 2


_YR = NODES * C // 128


def _prep_body(xg8_ref, wt_ref, wroott_ref, wrest_ref, b8_ref, bres8_ref,
               y_ref, root_ref, res_ref):
    k = pl.program_id(0)
    rb = lax.broadcasted_iota(jnp.int32, (128, 128), 0) // C
    cb = lax.broadcasted_iota(jnp.int32, (128, 128), 1) // C
    mask = (rb == cb).astype(jnp.float32)
    xg8 = xg8_ref[...]
    y_ref[...] = lax.dot_general(xg8, wt_ref[0] * mask,
                                 (((1,), (0,)), ((), ())),
                                 preferred_element_type=jnp.float32)

    @pl.when(k == NK - 1)
    def _():
        root_ref[...] = lax.dot_general(
            xg8, wroott_ref[...] * mask, (((1,), (0,)), ((), ())),
            preferred_element_type=jnp.float32) + b8_ref[...]
        rr = lax.dot_general(
            xg8, wrest_ref[...] * mask, (((1,), (0,)), ((), ())),
            preferred_element_type=jnp.float32) + bres8_ref[...]
        res_ref[...] = jnp.where(rr > 0, rr, jnp.exp(rr) - 1.0)


_prep = pl.pallas_call(
    _prep_body,
    grid=(NK,),
    in_specs=[
        pl.BlockSpec((_YR, 128), lambda k: (0, 0)),
        pl.BlockSpec((1, 128, 128), lambda k: (k, 0, 0)),
        pl.BlockSpec((128, 128), lambda k: (0, 0)),
        pl.BlockSpec((128, 128), lambda k: (0, 0)),
        pl.BlockSpec((1, 128), lambda k: (0, 0)),
        pl.BlockSpec((1, 128), lambda k: (0, 0)),
    ],
    out_specs=[
        pl.BlockSpec((_YR, 128), lambda k: (k, 0)),
        pl.BlockSpec((_YR, 128), lambda k: (0, 0)),
        pl.BlockSpec((_YR, 128), lambda k: (0, 0)),
    ],
    out_shape=(
        jax.ShapeDtypeStruct((NK * _YR, 128), jnp.float32),
        jax.ShapeDtypeStruct((_YR, 128), jnp.float32),
        jax.ShapeDtypeStruct((_YR, 128), jnp.float32),
    ),
)


def _lane_bcast(vec, lane):
    return lax.gather(
        vec, jnp.full((16, 1), lane, jnp.int32),
        lax.GatherDimensionNumbers(offset_dims=(), collapsed_slice_dims=(0,),
                                   start_index_map=(0,)),
        (1,), mode=lax.GatherScatterMode.PROMISE_IN_BOUNDS)


def _edges_body(y_hbm, ea0_hbm, ea1_hbm, src_hbm, dst_hbm, root_hbm,
                res_hbm, out_hbm,
                ia0, ia1, ia2, ia3, ib0, ib1, ib2, ib3, dst_v,
                bas0, bas1, bas2, bas3, ea0_v, ea1_v, src_v,
                ra0, ra1, ra2, ra3, rb0, rb1, rb2, rb3,
                m_v, agg_sp, cnt_sp, sem_a, sem_b):
    c = lax.axis_index("c")
    s = lax.axis_index("s")
    e0 = s * EPT
    rpt = GPS * V // 16

    idx_a = (ia0, ia1, ia2, ia3)
    idx_b = (ib0, ib1, ib2, ib3)
    rows_a = (ra0, ra1, ra2, ra3)
    rows_b = (rb0, rb1, rb2, rb3)

    @plsc.parallel_loop(0, rpt, step=1, unroll=4)
    def _fill(j):
        m_v[j] = jnp.zeros((C,), jnp.float32)
        ra0[j] = jnp.ones((C,), jnp.float32)
        ra0[j + rpt] = jnp.ones((C,), jnp.float32)

    pltpu.sync_copy(m_v.at[pl.ds(0, rpt)], agg_sp.at[pl.ds(s * rpt, rpt)])
    pltpu.sync_copy(m_v.at[pl.ds(0, V // 16)],
                    cnt_sp.at[pl.ds(s * (V // 16), V // 16)])

    pltpu.sync_copy(ea0_hbm.at[pl.ds(e0, EPT)], ea0_v)
    pltpu.sync_copy(ea1_hbm.at[pl.ds(e0, EPT)], ea1_v)
    pltpu.sync_copy(src_hbm.at[pl.ds(e0, EPT)], src_v)
    pltpu.sync_copy(dst_hbm.at[pl.ds(e0, EPT)], dst_v)

    off0 = c * (GPS * V)

    @plsc.parallel_loop(0, EPT // 16, step=1, unroll=2)
    def _setup(j):
        sl = pl.ds(j * 16, 16)
        p0 = ea0_v[sl] * (K - 1.0)
        p1 = ea1_v[sl] * (K - 1.0)
        i0 = jnp.minimum(p0.astype(jnp.int32), K - 2)
        i1 = jnp.minimum(p1.astype(jnp.int32), K - 2)
        f0 = p0 - i0.astype(jnp.float32)
        f1 = p1 - i1.astype(jnp.float32)
        g0 = 1.0 - f0
        g1 = 1.0 - f1
        bas0[sl] = g0 * g1
        bas1[sl] = f0 * g1
        bas2[sl] = g0 * f1
        bas3[sl] = f0 * f1
        wib = src_v[sl] + (i0 + i1 * K) * NODES + off0
        ia0[sl] = wib
        ia1[sl] = wib + NODES
        ia2[sl] = wib + K * NODES
        ia3[sl] = wib + (K + 1) * NODES

    def _addv(dref, sref, nchunks, val):
        @plsc.parallel_loop(0, nchunks, step=1, unroll=2)
        def f(j):
            dref[pl.ds(j * 16, 16)] = sref[pl.ds(j * 16, 16)] + val

    plsc.subcore_barrier()

    pltpu.sync_copy(ra0, cnt_sp.at[dst_v], add=True)

    bufs = ((idx_a, rows_a, sem_a), (idx_b, rows_b, sem_b))
    descs = [pltpu.async_copy(y_hbm.at[ix], r, sem_a)
             for ix, r in zip(idx_a, rows_a)]
    for g in range(GPS):
        cur_i, cur_r, _ = bufs[g % 2]
        nxt_i, nxt_r, nxt_s = bufs[(g + 1) % 2]
        if g < GPS - 1:
            with jax.named_scope("idxprep"):
                for corner in range(4):
                    _addv(nxt_i[corner], cur_i[corner], EPT // 16, V)
        with jax.named_scope("dwait"):
            for d in descs:
                d.wait()
        if g < GPS - 1:
            descs = [pltpu.async_copy(y_hbm.at[ix], r, nxt_s)
                     for ix, r in zip(nxt_i, nxt_r)]

        c0, c1, c2, c3 = cur_r

        with jax.named_scope("fma"):
            @plsc.parallel_loop(0, EPT // 16, step=1, unroll=1)
            def _body(ch):
                base = ch * 16
                sl = pl.ds(base, 16)
                b0 = bas0[sl]
                b1 = bas1[sl]
                b2v = bas2[sl]
                b3 = bas3[sl]
                for u in range(16):
                    ee = base + u
                    m = (c0[ee] * _lane_bcast(b0, u)
                         + c1[ee] * _lane_bcast(b1, u)
                         + c2[ee] * _lane_bcast(b2v, u)
                         + c3[ee] * _lane_bcast(b3, u))
                    m_v[ee] = m

        with jax.named_scope("scat"):
            pltpu.sync_copy(m_v, agg_sp.at[dst_v], add=True)
        if g < GPS - 1:
            with jax.named_scope("dstinc"):
                _addv(dst_v, dst_v, EPT // 16, V)

    plsc.subcore_barrier()

    row0 = s * rpt
    gbase = c * (GPS * V) + row0
    v_off = lax.rem(s, 2) * rpt
    pltpu.sync_copy(agg_sp.at[pl.ds(row0, rpt)], m_v.at[pl.ds(0, rpt)])
    pltpu.sync_copy(cnt_sp.at[pl.ds(v_off, rpt)], ra2.at[pl.ds(0, rpt)])
    pltpu.sync_copy(root_hbm.at[pl.ds(gbase, rpt)], ra0.at[pl.ds(0, rpt)])
    pltpu.sync_copy(res_hbm.at[pl.ds(gbase, rpt)], ra1.at[pl.ds(0, rpt)])

    def _fin(j, carry):
        cntv = jnp.maximum(ra2[j], 1.0)
        xo = m_v[j] / cntv + ra0[j]
        xo = jnp.where(xo > 0, xo, jnp.exp(xo) - 1.0)
        xo = xo + ra1[j]
        xo = jnp.where(xo > 0, xo, jnp.exp(xo) - 1.0)
        m_v[j + rpt] = xo
        return carry
    lax.fori_loop(0, rpt, _fin, 0)
    pltpu.sync_copy(m_v.at[pl.ds(rpt, rpt)], out_hbm.at[pl.ds(gbase, rpt)])


@functools.cache
def _get_edges():
    mesh = plsc.VectorSubcoreMesh(core_axis_name="c", subcore_axis_name="s",
                                  num_cores=2, num_subcores=16)
    idx_t = pltpu.VMEM((EPT,), jnp.int32)
    vec_t = pltpu.VMEM((EPT,), jnp.float32)
    row_t = pltpu.VMEM((EPT, C), jnp.float32)
    return pl.kernel(
        _edges_body,
        mesh=mesh,
        compiler_params=pltpu.CompilerParams(use_tc_tiling_on_sc=False),
        out_type=jax.ShapeDtypeStruct((NODES, C), jnp.float32),
        scratch_types=[
            idx_t, idx_t, idx_t, idx_t,
            idx_t, idx_t, idx_t, idx_t,
            idx_t,
            vec_t, vec_t, vec_t, vec_t,
            vec_t, vec_t,
            idx_t,
            row_t, row_t, row_t, row_t,
            row_t, row_t, row_t, row_t,
            row_t,
            pltpu.VMEM_SHARED((GPS * V, C), jnp.float32),
            pltpu.VMEM_SHARED((V, C), jnp.float32),
            pltpu.SemaphoreType.DMA,
            pltpu.SemaphoreType.DMA,
        ],
    )


def kernel(x, edge_index, edge_attr, Wspline, Wroot, b, Wres, bres):
    n, v, cc, t = x.shape
    xg = x.transpose(3, 0, 1, 2).reshape(NODES, C)

    ea0 = edge_attr[:NEB, 0]
    ea1 = edge_attr[:NEB, 1]
    src = edge_index[0, :NEB]
    dst = edge_index[1, :NEB]

    wt = jnp.tile(Wspline, (1, 8, 8))
    wroott = jnp.tile(Wroot, (8, 8))
    wrest = jnp.tile(Wres.T, (8, 8))
    b8 = jnp.tile(b, 8).reshape(1, 128)
    bres8 = jnp.tile(bres, 8).reshape(1, 128)
    xg128 = xg.reshape(_YR, 128)

    y, root128, res128 = _prep(xg128, wt, wroott, wrest, b8, bres8)
    yflat = y.reshape(NK * NODES, C)
    root = root128.reshape(NODES, C)
    res = res128.reshape(NODES, C)

    out_node = _get_edges()(yflat, ea0, ea1, src, dst, root, res)

    return out_node.reshape(t, n, v, C).transpose(1, 2, 3, 0)


# speedup vs baseline: 1.2833x; 1.1942x over previous
import functools

import jax
import jax.numpy as jnp
from jax import lax
from jax.experimental import pallas as pl
from jax.experimental.pallas import tpu as pltpu
from jax.experimental.pallas import tpu_sc as plsc

K = 5
V = 512
C = 16
NG = 16
NEB = 8192
NODES = NG * V
NK = K * K
EPT = NEB // 16
GPS = NG // 2


_YR = NODES * C // 128


def _mm(a, bm):
    return lax.dot_general(a, bm, (((1,), (0,)), ((), ())),
                           preferred_element_type=jnp.float32)


def _prep_body(xg8_ref, wsp_ref, wroot_ref, wrest_ref, b2_ref, bres2_ref,
               e8_ref, e8t_ref, y_ref, root_ref, res_ref):
    k = pl.program_id(0)
    rb = lax.broadcasted_iota(jnp.int32, (128, 128), 0) // C
    cb = lax.broadcasted_iota(jnp.int32, (128, 128), 1) // C
    mask = (rb == cb).astype(jnp.float32)
    e8 = e8_ref[...]
    e8t = e8t_ref[...]
    xg8 = xg8_ref[...]

    def blkdiag(w16):
        return _mm(_mm(e8, w16), e8t) * mask

    y_ref[...] = _mm(xg8, blkdiag(wsp_ref[0]))

    @pl.when(k == NK - 1)
    def _():
        b8 = _mm(b2_ref[...], e8t)
        bres8 = _mm(bres2_ref[...], e8t)
        root_ref[...] = _mm(xg8, blkdiag(wroot_ref[...])) + b8
        rr = _mm(xg8, blkdiag(wrest_ref[...])) + bres8
        res_ref[...] = jnp.where(rr > 0, rr, jnp.exp(rr) - 1.0)


_prep = pl.pallas_call(
    _prep_body,
    grid=(NK,),
    in_specs=[
        pl.BlockSpec((_YR, 128), lambda k: (0, 0)),
        pl.BlockSpec((1, C, C), lambda k: (k, 0, 0)),
        pl.BlockSpec((C, C), lambda k: (0, 0)),
        pl.BlockSpec((C, C), lambda k: (0, 0)),
        pl.BlockSpec((1, C), lambda k: (0, 0)),
        pl.BlockSpec((1, C), lambda k: (0, 0)),
        pl.BlockSpec((128, C), lambda k: (0, 0)),
        pl.BlockSpec((C, 128), lambda k: (0, 0)),
    ],
    out_specs=[
        pl.BlockSpec((_YR, 128), lambda k: (k, 0)),
        pl.BlockSpec((_YR, 128), lambda k: (0, 0)),
        pl.BlockSpec((_YR, 128), lambda k: (0, 0)),
    ],
    out_shape=(
        jax.ShapeDtypeStruct((NK * _YR, 128), jnp.float32),
        jax.ShapeDtypeStruct((_YR, 128), jnp.float32),
        jax.ShapeDtypeStruct((_YR, 128), jnp.float32),
    ),
)


def _lane_bcast(vec, lane):
    return lax.gather(
        vec, jnp.full((16, 1), lane, jnp.int32),
        lax.GatherDimensionNumbers(offset_dims=(), collapsed_slice_dims=(0,),
                                   start_index_map=(0,)),
        (1,), mode=lax.GatherScatterMode.PROMISE_IN_BOUNDS)


def _edges_body(y_hbm, ea0_hbm, ea1_hbm, src_hbm, dst_hbm, root_hbm,
                res_hbm, out_hbm,
                ia0, ia1, ia2, ia3, ib0, ib1, ib2, ib3, dst_v,
                bas0, bas1, bas2, bas3, ea0_v, ea1_v, src_v,
                ra0, ra1, ra2, ra3, rb0, rb1, rb2, rb3,
                m_v, agg_sp, cnt_sp, sem_a, sem_b):
    c = lax.axis_index("c")
    s = lax.axis_index("s")
    e0 = s * EPT
    rpt = GPS * V // 16

    idx_a = (ia0, ia1, ia2, ia3)
    idx_b = (ib0, ib1, ib2, ib3)
    rows_a = (ra0, ra1, ra2, ra3)
    rows_b = (rb0, rb1, rb2, rb3)

    @plsc.parallel_loop(0, rpt, step=1, unroll=4)
    def _fill(j):
        m_v[j] = jnp.zeros((C,), jnp.float32)
        ra0[j] = jnp.ones((C,), jnp.float32)
        ra0[j + rpt] = jnp.ones((C,), jnp.float32)

    pltpu.sync_copy(m_v.at[pl.ds(0, rpt)], agg_sp.at[pl.ds(s * rpt, rpt)])
    pltpu.sync_copy(m_v.at[pl.ds(0, V // 16)],
                    cnt_sp.at[pl.ds(s * (V // 16), V // 16)])

    pltpu.sync_copy(ea0_hbm.at[pl.ds(e0, EPT)], ea0_v)
    pltpu.sync_copy(ea1_hbm.at[pl.ds(e0, EPT)], ea1_v)
    pltpu.sync_copy(src_hbm.at[pl.ds(e0, EPT)], src_v)
    pltpu.sync_copy(dst_hbm.at[pl.ds(e0, EPT)], dst_v)

    off0 = c * (GPS * V)

    @plsc.parallel_loop(0, EPT // 16, step=1, unroll=2)
    def _setup(j):
        sl = pl.ds(j * 16, 16)
        p0 = ea0_v[sl] * (K - 1.0)
        p1 = ea1_v[sl] * (K - 1.0)
        i0 = jnp.minimum(p0.astype(jnp.int32), K - 2)
        i1 = jnp.minimum(p1.astype(jnp.int32), K - 2)
        f0 = p0 - i0.astype(jnp.float32)
        f1 = p1 - i1.astype(jnp.float32)
        g0 = 1.0 - f0
        g1 = 1.0 - f1
        bas0[sl] = g0 * g1
        bas1[sl] = f0 * g1
        bas2[sl] = g0 * f1
        bas3[sl] = f0 * f1
        wib = src_v[sl] + (i0 + i1 * K) * NODES + off0
        ia0[sl] = wib
        ia1[sl] = wib + NODES
        ia2[sl] = wib + K * NODES
        ia3[sl] = wib + (K + 1) * NODES

    def _addv(dref, sref, nchunks, val):
        @plsc.parallel_loop(0, nchunks, step=1, unroll=2)
        def f(j):
            dref[pl.ds(j * 16, 16)] = sref[pl.ds(j * 16, 16)] + val

    plsc.subcore_barrier()

    pltpu.sync_copy(ra0, cnt_sp.at[dst_v], add=True)

    bufs = ((idx_a, rows_a, sem_a), (idx_b, rows_b, sem_b))
    descs = [pltpu.async_copy(y_hbm.at[ix], r, sem_a)
             for ix, r in zip(idx_a, rows_a)]
    for g in range(GPS):
        cur_i, cur_r, _ = bufs[g % 2]
        nxt_i, nxt_r, nxt_s = bufs[(g + 1) % 2]
        if g < GPS - 1:
            with jax.named_scope("idxprep"):
                for corner in range(4):
                    _addv(nxt_i[corner], cur_i[corner], EPT // 16, V)
        with jax.named_scope("dwait"):
            for d in descs:
                d.wait()
        if g < GPS - 1:
            descs = [pltpu.async_copy(y_hbm.at[ix], r, nxt_s)
                     for ix, r in zip(nxt_i, nxt_r)]

        c0, c1, c2, c3 = cur_r

        with jax.named_scope("fma"):
            @plsc.parallel_loop(0, EPT // 16, step=1, unroll=1)
            def _body(ch):
                base = ch * 16
                sl = pl.ds(base, 16)
                b0 = bas0[sl]
                b1 = bas1[sl]
                b2v = bas2[sl]
                b3 = bas3[sl]
                for u in range(16):
                    ee = base + u
                    m = (c0[ee] * _lane_bcast(b0, u)
                         + c1[ee] * _lane_bcast(b1, u)
                         + c2[ee] * _lane_bcast(b2v, u)
                         + c3[ee] * _lane_bcast(b3, u))
                    m_v[ee] = m

        with jax.named_scope("scat"):
            pltpu.sync_copy(m_v, agg_sp.at[dst_v], add=True)
        if g < GPS - 1:
            with jax.named_scope("dstinc"):
                _addv(dst_v, dst_v, EPT // 16, V)

    plsc.subcore_barrier()

    row0 = s * rpt
    gbase = c * (GPS * V) + row0
    v_off = lax.rem(s, 2) * rpt
    pltpu.sync_copy(agg_sp.at[pl.ds(row0, rpt)], m_v.at[pl.ds(0, rpt)])
    pltpu.sync_copy(cnt_sp.at[pl.ds(v_off, rpt)], ra2.at[pl.ds(0, rpt)])
    pltpu.sync_copy(root_hbm.at[pl.ds(gbase, rpt)], ra0.at[pl.ds(0, rpt)])
    pltpu.sync_copy(res_hbm.at[pl.ds(gbase, rpt)], ra1.at[pl.ds(0, rpt)])

    def _fin(j, carry):
        cntv = jnp.maximum(ra2[j], 1.0)
        xo = m_v[j] / cntv + ra0[j]
        xo = jnp.where(xo > 0, xo, jnp.exp(xo) - 1.0)
        xo = xo + ra1[j]
        xo = jnp.where(xo > 0, xo, jnp.exp(xo) - 1.0)
        m_v[j + rpt] = xo
        return carry
    lax.fori_loop(0, rpt, _fin, 0)
    pltpu.sync_copy(m_v.at[pl.ds(rpt, rpt)], out_hbm.at[pl.ds(gbase, rpt)])


@functools.cache
def _get_edges():
    mesh = plsc.VectorSubcoreMesh(core_axis_name="c", subcore_axis_name="s",
                                  num_cores=2, num_subcores=16)
    idx_t = pltpu.VMEM((EPT,), jnp.int32)
    vec_t = pltpu.VMEM((EPT,), jnp.float32)
    row_t = pltpu.VMEM((EPT, C), jnp.float32)
    return pl.kernel(
        _edges_body,
        mesh=mesh,
        compiler_params=pltpu.CompilerParams(use_tc_tiling_on_sc=False),
        out_type=jax.ShapeDtypeStruct((NODES, C), jnp.float32),
        scratch_types=[
            idx_t, idx_t, idx_t, idx_t,
            idx_t, idx_t, idx_t, idx_t,
            idx_t,
            vec_t, vec_t, vec_t, vec_t,
            vec_t, vec_t,
            idx_t,
            row_t, row_t, row_t, row_t,
            row_t, row_t, row_t, row_t,
            row_t,
            pltpu.VMEM_SHARED((GPS * V, C), jnp.float32),
            pltpu.VMEM_SHARED((V, C), jnp.float32),
            pltpu.SemaphoreType.DMA,
            pltpu.SemaphoreType.DMA,
        ],
    )


def kernel(x, edge_index, edge_attr, Wspline, Wroot, b, Wres, bres):
    n, v, cc, t = x.shape
    xg = x.transpose(3, 0, 1, 2).reshape(NODES, C)

    ea0 = edge_attr[:NEB, 0]
    ea1 = edge_attr[:NEB, 1]
    src = edge_index[0, :NEB]
    dst = edge_index[1, :NEB]

    e8 = jnp.tile(jnp.eye(C, dtype=jnp.float32), (8, 1))
    b2 = b.reshape(1, C)
    bres2 = bres.reshape(1, C)
    xg128 = xg.reshape(_YR, 128)

    y, root128, res128 = _prep(xg128, Wspline, Wroot, Wres.T, b2, bres2,
                               e8, e8.T)
    yflat = y.reshape(NK * NODES, C)
    root = root128.reshape(NODES, C)
    res = res128.reshape(NODES, C)

    out_node = _get_edges()(yflat, ea0, ea1, src, dst, root, res)

    return out_node.reshape(t, n, v, C).transpose(1, 2, 3, 0)


# speedup vs baseline: 1.2847x; 1.0011x over previous
import functools

import jax
import jax.numpy as jnp
from jax import lax
from jax.experimental import pallas as pl
from jax.experimental.pallas import tpu as pltpu
from jax.experimental.pallas import tpu_sc as plsc

K = 5
V = 512
C = 16
NG = 16
NEB = 8192
NODES = NG * V
NK = K * K
EPT = NEB // 16
GPS = NG // 2


_YR = NODES * C // 128


def _mm(a, bm):
    return lax.dot_general(a, bm, (((1,), (0,)), ((), ())),
                           preferred_element_type=jnp.float32)


def _prep_body(xg8_ref, wsp_ref, wroot_ref, wrest_ref, b2_ref, bres2_ref,
               e8_ref, e8t_ref, y_ref, root_ref, res_ref):
    k = pl.program_id(0)
    rb = lax.broadcasted_iota(jnp.int32, (128, 128), 0) // C
    cb = lax.broadcasted_iota(jnp.int32, (128, 128), 1) // C
    mask = (rb == cb).astype(jnp.float32)
    e8 = e8_ref[...]
    e8t = e8t_ref[...]
    xg8 = xg8_ref[...]

    def blkdiag(w16):
        return _mm(_mm(e8, w16), e8t) * mask

    y_ref[...] = _mm(xg8, blkdiag(wsp_ref[0]))

    @pl.when(k == NK - 1)
    def _():
        b8 = _mm(b2_ref[...], e8t)
        bres8 = _mm(bres2_ref[...], e8t)
        root_ref[...] = _mm(xg8, blkdiag(wroot_ref[...])) + b8
        rr = _mm(xg8, blkdiag(wrest_ref[...])) + bres8
        res_ref[...] = jnp.where(rr > 0, rr, jnp.exp(rr) - 1.0)


_prep = pl.pallas_call(
    _prep_body,
    grid=(NK,),
    in_specs=[
        pl.BlockSpec((_YR, 128), lambda k: (0, 0)),
        pl.BlockSpec((1, C, C), lambda k: (k, 0, 0)),
        pl.BlockSpec((C, C), lambda k: (0, 0)),
        pl.BlockSpec((C, C), lambda k: (0, 0)),
        pl.BlockSpec((1, C), lambda k: (0, 0)),
        pl.BlockSpec((1, C), lambda k: (0, 0)),
        pl.BlockSpec((128, C), lambda k: (0, 0)),
        pl.BlockSpec((C, 128), lambda k: (0, 0)),
    ],
    out_specs=[
        pl.BlockSpec((_YR, 128), lambda k: (k, 0)),
        pl.BlockSpec((_YR, 128), lambda k: (0, 0)),
        pl.BlockSpec((_YR, 128), lambda k: (0, 0)),
    ],
    out_shape=(
        jax.ShapeDtypeStruct((NK * _YR, 128), jnp.float32),
        jax.ShapeDtypeStruct((_YR, 128), jnp.float32),
        jax.ShapeDtypeStruct((_YR, 128), jnp.float32),
    ),
)


def _lane_bcast(vec, lane):
    return lax.gather(
        vec, jnp.full((16, 1), lane, jnp.int32),
        lax.GatherDimensionNumbers(offset_dims=(), collapsed_slice_dims=(0,),
                                   start_index_map=(0,)),
        (1,), mode=lax.GatherScatterMode.PROMISE_IN_BOUNDS)


def _edges_body(y_hbm, ea0_hbm, ea1_hbm, src_hbm, dst_hbm, root_hbm,
                res_hbm, out_hbm,
                ia0, ia1, ia2, ia3, ib0, ib1, ib2, ib3, dst_v,
                bas0, bas1, bas2, bas3, ea0_v, ea1_v, src_v,
                ra0, ra1, ra2, ra3, rb0, rb1, rb2, rb3,
                m_v, agg_sp, cnt_sp, sem_a, sem_b):
    c = lax.axis_index("c")
    s = lax.axis_index("s")
    e0 = s * EPT
    rpt = GPS * V // 16

    idx_a = (ia0, ia1, ia2, ia3)
    idx_b = (ib0, ib1, ib2, ib3)
    rows_a = (ra0, ra1, ra2, ra3)
    rows_b = (rb0, rb1, rb2, rb3)

    @plsc.parallel_loop(0, rpt, step=1, unroll=4)
    def _fill(j):
        m_v[j] = jnp.zeros((C,), jnp.float32)
        ra0[j] = jnp.ones((C,), jnp.float32)
        ra0[j + rpt] = jnp.ones((C,), jnp.float32)

    pltpu.sync_copy(m_v.at[pl.ds(0, rpt)], agg_sp.at[pl.ds(s * rpt, rpt)])
    pltpu.sync_copy(m_v.at[pl.ds(0, V // 16)],
                    cnt_sp.at[pl.ds(s * (V // 16), V // 16)])

    pltpu.sync_copy(ea0_hbm.at[pl.ds(e0, EPT)], ea0_v)
    pltpu.sync_copy(ea1_hbm.at[pl.ds(e0, EPT)], ea1_v)
    pltpu.sync_copy(src_hbm.at[pl.ds(e0, EPT)], src_v)
    pltpu.sync_copy(dst_hbm.at[pl.ds(e0, EPT)], dst_v)

    off0 = c * (GPS * V)

    @plsc.parallel_loop(0, EPT // 16, step=1, unroll=2)
    def _setup(j):
        sl = pl.ds(j * 16, 16)
        p0 = ea0_v[sl] * (K - 1.0)
        p1 = ea1_v[sl] * (K - 1.0)
        i0 = jnp.minimum(p0.astype(jnp.int32), K - 2)
        i1 = jnp.minimum(p1.astype(jnp.int32), K - 2)
        f0 = p0 - i0.astype(jnp.float32)
        f1 = p1 - i1.astype(jnp.float32)
        g0 = 1.0 - f0
        g1 = 1.0 - f1
        bas0[sl] = g0 * g1
        bas1[sl] = f0 * g1
        bas2[sl] = g0 * f1
        bas3[sl] = f0 * f1
        wib = src_v[sl] + (i0 + i1 * K) * NODES + off0
        ia0[sl] = wib
        ia1[sl] = wib + NODES
        ia2[sl] = wib + K * NODES
        ia3[sl] = wib + (K + 1) * NODES

    def _addv(dref, sref, nchunks, val):
        @plsc.parallel_loop(0, nchunks, step=1, unroll=2)
        def f(j):
            dref[pl.ds(j * 16, 16)] = sref[pl.ds(j * 16, 16)] + val

    plsc.subcore_barrier()

    pltpu.sync_copy(ra0, cnt_sp.at[dst_v], add=True)

    bufs = ((idx_a, rows_a, sem_a), (idx_b, rows_b, sem_b))
    descs = [pltpu.async_copy(y_hbm.at[ix], r, sem_a)
             for ix, r in zip(idx_a, rows_a)]
    for g in range(GPS):
        cur_i, cur_r, _ = bufs[g % 2]
        nxt_i, nxt_r, nxt_s = bufs[(g + 1) % 2]
        if g < GPS - 1:
            with jax.named_scope("idxprep"):
                for corner in range(4):
                    _addv(nxt_i[corner], cur_i[corner], EPT // 16, V)
        cur_descs = descs
        if g < GPS - 1:
            descs = [pltpu.async_copy(y_hbm.at[ix], r, nxt_s)
                     for ix, r in zip(nxt_i, nxt_r)]
        with jax.named_scope("dwait"):
            for d in cur_descs:
                d.wait()

        c0, c1, c2, c3 = cur_r

        with jax.named_scope("fma"):
            @plsc.parallel_loop(0, EPT // 16, step=1, unroll=1)
            def _body(ch):
                base = ch * 16
                sl = pl.ds(base, 16)
                b0 = bas0[sl]
                b1 = bas1[sl]
                b2v = bas2[sl]
                b3 = bas3[sl]
                for u in range(16):
                    ee = base + u
                    m = (c0[ee] * _lane_bcast(b0, u)
                         + c1[ee] * _lane_bcast(b1, u)
                         + c2[ee] * _lane_bcast(b2v, u)
                         + c3[ee] * _lane_bcast(b3, u))
                    m_v[ee] = m

        with jax.named_scope("scat"):
            pltpu.sync_copy(m_v, agg_sp.at[dst_v], add=True)
        if g < GPS - 1:
            with jax.named_scope("dstinc"):
                _addv(dst_v, dst_v, EPT // 16, V)

    plsc.subcore_barrier()

    row0 = s * rpt
    gbase = c * (GPS * V) + row0
    v_off = lax.rem(s, 2) * rpt
    pltpu.sync_copy(agg_sp.at[pl.ds(row0, rpt)], m_v.at[pl.ds(0, rpt)])
    pltpu.sync_copy(cnt_sp.at[pl.ds(v_off, rpt)], ra2.at[pl.ds(0, rpt)])
    pltpu.sync_copy(root_hbm.at[pl.ds(gbase, rpt)], ra0.at[pl.ds(0, rpt)])
    pltpu.sync_copy(res_hbm.at[pl.ds(gbase, rpt)], ra1.at[pl.ds(0, rpt)])

    def _fin(j, carry):
        cntv = jnp.maximum(ra2[j], 1.0)
        xo = m_v[j] / cntv + ra0[j]
        xo = jnp.where(xo > 0, xo, jnp.exp(xo) - 1.0)
        xo = xo + ra1[j]
        xo = jnp.where(xo > 0, xo, jnp.exp(xo) - 1.0)
        m_v[j + rpt] = xo
        return carry
    lax.fori_loop(0, rpt, _fin, 0)
    pltpu.sync_copy(m_v.at[pl.ds(rpt, rpt)], out_hbm.at[pl.ds(gbase, rpt)])


@functools.cache
def _get_edges():
    mesh = plsc.VectorSubcoreMesh(core_axis_name="c", subcore_axis_name="s",
                                  num_cores=2, num_subcores=16)
    idx_t = pltpu.VMEM((EPT,), jnp.int32)
    vec_t = pltpu.VMEM((EPT,), jnp.float32)
    row_t = pltpu.VMEM((EPT, C), jnp.float32)
    return pl.kernel(
        _edges_body,
        mesh=mesh,
        compiler_params=pltpu.CompilerParams(use_tc_tiling_on_sc=False),
        out_type=jax.ShapeDtypeStruct((NODES, C), jnp.float32),
        scratch_types=[
            idx_t, idx_t, idx_t, idx_t,
            idx_t, idx_t, idx_t, idx_t,
            idx_t,
            vec_t, vec_t, vec_t, vec_t,
            vec_t, vec_t,
            idx_t,
            row_t, row_t, row_t, row_t,
            row_t, row_t, row_t, row_t,
            row_t,
            pltpu.VMEM_SHARED((GPS * V, C), jnp.float32),
            pltpu.VMEM_SHARED((V, C), jnp.float32),
            pltpu.SemaphoreType.DMA,
            pltpu.SemaphoreType.DMA,
        ],
    )


def kernel(x, edge_index, edge_attr, Wspline, Wroot, b, Wres, bres):
    n, v, cc, t = x.shape
    xg = x.transpose(3, 0, 1, 2).reshape(NODES, C)

    ea0 = edge_attr[:NEB, 0]
    ea1 = edge_attr[:NEB, 1]
    src = edge_index[0, :NEB]
    dst = edge_index[1, :NEB]

    e8 = jnp.tile(jnp.eye(C, dtype=jnp.float32), (8, 1))
    b2 = b.reshape(1, C)
    bres2 = bres.reshape(1, C)
    xg128 = xg.reshape(_YR, 128)

    y, root128, res128 = _prep(xg128, Wspline, Wroot, Wres.T, b2, bres2,
                               e8, e8.T)
    yflat = y.reshape(NK * NODES, C)
    root = root128.reshape(NODES, C)
    res = res128.reshape(NODES, C)

    out_node = _get_edges()(yflat, ea0, ea1, src, dst, root, res)

    return out_node.reshape(t, n, v, C).transpose(1, 2, 3, 0)


# speedup vs baseline: 1.3116x; 1.0210x over previous
import functools

import jax
import jax.numpy as jnp
from jax import lax
from jax.experimental import pallas as pl
from jax.experimental.pallas import tpu as pltpu
from jax.experimental.pallas import tpu_sc as plsc

K = 5
V = 512
C = 16
NG = 16
NEB = 8192
NODES = NG * V
NK = K * K
EPT = NEB // 16
GPS = NG // 2


_YR = NODES * C // 128


def _mm(a, bm, algo=None):
    return lax.dot_general(a, bm, (((1,), (0,)), ((), ())),
                           precision=algo,
                           preferred_element_type=jnp.float32)


def _prep_body(xg8_ref, wsp_ref, wroot_ref, wrest_ref, b2_ref, bres2_ref,
               e8_ref, e8t_ref, y_ref, root_ref, res_ref):
    k = pl.program_id(0)
    rb = lax.broadcasted_iota(jnp.int32, (128, 128), 0) // C
    cb = lax.broadcasted_iota(jnp.int32, (128, 128), 1) // C
    mask = (rb == cb).astype(jnp.float32)
    e8t = e8t_ref[...]
    xg8 = xg8_ref[...]

    def blkdiag(w16):
        return pltpu.repeat(pltpu.repeat(w16, 8, 0), 8, 1) * mask

    y_ref[...] = _mm(xg8, blkdiag(wsp_ref[0]))

    @pl.when(k == NK - 1)
    def _():
        b8 = _mm(b2_ref[...], e8t)
        bres8 = _mm(bres2_ref[...], e8t)
        root_ref[...] = _mm(xg8, blkdiag(wroot_ref[...])) + b8
        rr = _mm(xg8, blkdiag(wrest_ref[...])) + bres8
        res_ref[...] = jnp.where(rr > 0, rr, jnp.exp(rr) - 1.0)


_prep = pl.pallas_call(
    _prep_body,
    grid=(NK,),
    in_specs=[
        pl.BlockSpec((_YR, 128), lambda k: (0, 0)),
        pl.BlockSpec((1, C, C), lambda k: (k, 0, 0)),
        pl.BlockSpec((C, C), lambda k: (0, 0)),
        pl.BlockSpec((C, C), lambda k: (0, 0)),
        pl.BlockSpec((1, C), lambda k: (0, 0)),
        pl.BlockSpec((1, C), lambda k: (0, 0)),
        pl.BlockSpec((128, C), lambda k: (0, 0)),
        pl.BlockSpec((C, 128), lambda k: (0, 0)),
    ],
    out_specs=[
        pl.BlockSpec((_YR, 128), lambda k: (k, 0)),
        pl.BlockSpec((_YR, 128), lambda k: (0, 0)),
        pl.BlockSpec((_YR, 128), lambda k: (0, 0)),
    ],
    out_shape=(
        jax.ShapeDtypeStruct((NK * _YR, 128), jnp.float32),
        jax.ShapeDtypeStruct((_YR, 128), jnp.float32),
        jax.ShapeDtypeStruct((_YR, 128), jnp.float32),
    ),
)


def _lane_bcast(vec, lane):
    return lax.gather(
        vec, jnp.full((16, 1), lane, jnp.int32),
        lax.GatherDimensionNumbers(offset_dims=(), collapsed_slice_dims=(0,),
                                   start_index_map=(0,)),
        (1,), mode=lax.GatherScatterMode.PROMISE_IN_BOUNDS)


def _edges_body(y_hbm, ea0_hbm, ea1_hbm, src_hbm, dst_hbm, root_hbm,
                res_hbm, out_hbm,
                ia0, ia1, ia2, ia3, ib0, ib1, ib2, ib3, dst_v,
                bas0, bas1, bas2, bas3, ea0_v, ea1_v, src_v,
                ra0, ra1, ra2, ra3, rb0, rb1, rb2, rb3,
                m_v, agg_sp, cnt_sp, sem_a, sem_b):
    c = lax.axis_index("c")
    s = lax.axis_index("s")
    e0 = s * EPT
    rpt = GPS * V // 16

    idx_a = (ia0, ia1, ia2, ia3)
    idx_b = (ib0, ib1, ib2, ib3)
    rows_a = (ra0, ra1, ra2, ra3)
    rows_b = (rb0, rb1, rb2, rb3)

    @plsc.parallel_loop(0, rpt, step=1, unroll=4)
    def _fill(j):
        m_v[j] = jnp.zeros((C,), jnp.float32)
        ra0[j] = jnp.ones((C,), jnp.float32)
        ra0[j + rpt] = jnp.ones((C,), jnp.float32)

    pltpu.sync_copy(m_v.at[pl.ds(0, rpt)], agg_sp.at[pl.ds(s * rpt, rpt)])
    pltpu.sync_copy(m_v.at[pl.ds(0, V // 16)],
                    cnt_sp.at[pl.ds(s * (V // 16), V // 16)])

    pltpu.sync_copy(ea0_hbm.at[pl.ds(e0, EPT)], ea0_v)
    pltpu.sync_copy(ea1_hbm.at[pl.ds(e0, EPT)], ea1_v)
    pltpu.sync_copy(src_hbm.at[pl.ds(e0, EPT)], src_v)
    pltpu.sync_copy(dst_hbm.at[pl.ds(e0, EPT)], dst_v)

    off0 = c * (GPS * V)

    @plsc.parallel_loop(0, EPT // 16, step=1, unroll=2)
    def _setup(j):
        sl = pl.ds(j * 16, 16)
        p0 = ea0_v[sl] * (K - 1.0)
        p1 = ea1_v[sl] * (K - 1.0)
        i0 = jnp.minimum(p0.astype(jnp.int32), K - 2)
        i1 = jnp.minimum(p1.astype(jnp.int32), K - 2)
        f0 = p0 - i0.astype(jnp.float32)
        f1 = p1 - i1.astype(jnp.float32)
        g0 = 1.0 - f0
        g1 = 1.0 - f1
        bas0[sl] = g0 * g1
        bas1[sl] = f0 * g1
        bas2[sl] = g0 * f1
        bas3[sl] = f0 * f1
        wib = src_v[sl] + (i0 + i1 * K) * NODES + off0
        ia0[sl] = wib
        ia1[sl] = wib + NODES
        ia2[sl] = wib + K * NODES
        ia3[sl] = wib + (K + 1) * NODES

    def _addv(dref, sref, nchunks, val):
        @plsc.parallel_loop(0, nchunks, step=1, unroll=2)
        def f(j):
            dref[pl.ds(j * 16, 16)] = sref[pl.ds(j * 16, 16)] + val

    plsc.subcore_barrier()

    pltpu.sync_copy(ra0, cnt_sp.at[dst_v], add=True)

    bufs = ((idx_a, rows_a, sem_a), (idx_b, rows_b, sem_b))
    descs = [pltpu.async_copy(y_hbm.at[ix], r, sem_a)
             for ix, r in zip(idx_a, rows_a)]
    for g in range(GPS):
        cur_i, cur_r, _ = bufs[g % 2]
        nxt_i, nxt_r, nxt_s = bufs[(g + 1) % 2]
        if g < GPS - 1:
            with jax.named_scope("idxprep"):
                for corner in range(4):
                    _addv(nxt_i[corner], cur_i[corner], EPT // 16, V)
        cur_descs = descs
        if g < GPS - 1:
            descs = [pltpu.async_copy(y_hbm.at[ix], r, nxt_s)
                     for ix, r in zip(nxt_i, nxt_r)]
        with jax.named_scope("dwait"):
            for d in cur_descs:
                d.wait()

        c0, c1, c2, c3 = cur_r

        with jax.named_scope("fma"):
            @plsc.parallel_loop(0, EPT // 16, step=1, unroll=1)
            def _body(ch):
                base = ch * 16
                sl = pl.ds(base, 16)
                b0 = bas0[sl]
                b1 = bas1[sl]
                b2v = bas2[sl]
                b3 = bas3[sl]
                for u in range(16):
                    ee = base + u
                    m = (c0[ee] * _lane_bcast(b0, u)
                         + c1[ee] * _lane_bcast(b1, u)
                         + c2[ee] * _lane_bcast(b2v, u)
                         + c3[ee] * _lane_bcast(b3, u))
                    m_v[ee] = m

        with jax.named_scope("scat"):
            pltpu.sync_copy(m_v, agg_sp.at[dst_v], add=True)
        if g < GPS - 1:
            with jax.named_scope("dstinc"):
                _addv(dst_v, dst_v, EPT // 16, V)

    plsc.subcore_barrier()

    row0 = s * rpt
    gbase = c * (GPS * V) + row0
    v_off = lax.rem(s, 2) * rpt
    pltpu.sync_copy(agg_sp.at[pl.ds(row0, rpt)], m_v.at[pl.ds(0, rpt)])
    pltpu.sync_copy(cnt_sp.at[pl.ds(v_off, rpt)], ra2.at[pl.ds(0, rpt)])
    pltpu.sync_copy(root_hbm.at[pl.ds(gbase, rpt)], ra0.at[pl.ds(0, rpt)])
    pltpu.sync_copy(res_hbm.at[pl.ds(gbase, rpt)], ra1.at[pl.ds(0, rpt)])

    def _fin(j, carry):
        cntv = jnp.maximum(ra2[j], 1.0)
        xo = m_v[j] / cntv + ra0[j]
        xo = jnp.where(xo > 0, xo, jnp.exp(xo) - 1.0)
        xo = xo + ra1[j]
        xo = jnp.where(xo > 0, xo, jnp.exp(xo) - 1.0)
        m_v[j + rpt] = xo
        return carry
    lax.fori_loop(0, rpt, _fin, 0)
    pltpu.sync_copy(m_v.at[pl.ds(rpt, rpt)], out_hbm.at[pl.ds(gbase, rpt)])


@functools.cache
def _get_edges():
    mesh = plsc.VectorSubcoreMesh(core_axis_name="c", subcore_axis_name="s",
                                  num_cores=2, num_subcores=16)
    idx_t = pltpu.VMEM((EPT,), jnp.int32)
    vec_t = pltpu.VMEM((EPT,), jnp.float32)
    row_t = pltpu.VMEM((EPT, C), jnp.float32)
    return pl.kernel(
        _edges_body,
        mesh=mesh,
        compiler_params=pltpu.CompilerParams(use_tc_tiling_on_sc=False),
        out_type=jax.ShapeDtypeStruct((NODES, C), jnp.float32),
        scratch_types=[
            idx_t, idx_t, idx_t, idx_t,
            idx_t, idx_t, idx_t, idx_t,
            idx_t,
            vec_t, vec_t, vec_t, vec_t,
            vec_t, vec_t,
            idx_t,
            row_t, row_t, row_t, row_t,
            row_t, row_t, row_t, row_t,
            row_t,
            pltpu.VMEM_SHARED((GPS * V, C), jnp.float32),
            pltpu.VMEM_SHARED((V, C), jnp.float32),
            pltpu.SemaphoreType.DMA,
            pltpu.SemaphoreType.DMA,
        ],
    )


def kernel(x, edge_index, edge_attr, Wspline, Wroot, b, Wres, bres):
    n, v, cc, t = x.shape
    xg = x.transpose(3, 0, 1, 2).reshape(NODES, C)

    ea0 = edge_attr[:NEB, 0]
    ea1 = edge_attr[:NEB, 1]
    src = edge_index[0, :NEB]
    dst = edge_index[1, :NEB]

    e8 = jnp.tile(jnp.eye(C, dtype=jnp.float32), (8, 1))
    b2 = b.reshape(1, C)
    bres2 = bres.reshape(1, C)
    xg128 = xg.reshape(_YR, 128)

    y, root128, res128 = _prep(xg128, Wspline, Wroot, Wres.T, b2, bres2,
                               e8, e8.T)
    yflat = y.reshape(NK * NODES, C)
    root = root128.reshape(NODES, C)
    res = res128.reshape(NODES, C)

    out_node = _get_edges()(yflat, ea0, ea1, src, dst, root, res)

    return out_node.reshape(t, n, v, C).transpose(1, 2, 3, 0)


# speedup vs baseline: 1.3117x; 1.0001x over previous
import functools

import jax
import jax.numpy as jnp
from jax import lax
from jax.experimental import pallas as pl
from jax.experimental.pallas import tpu as pltpu
from jax.experimental.pallas import tpu_sc as plsc

K = 5
V = 512
C = 16
NG = 16
NEB = 8192
NODES = NG * V
NK = K * K
EPT = NEB // 16
GPS = NG // 2


_YR = NODES * C // 128


def _mm(a, bm, algo=None):
    return lax.dot_general(a, bm, (((1,), (0,)), ((), ())),
                           precision=algo,
                           preferred_element_type=jnp.float32)


def _prep_body(xg8_ref, wsp_ref, wroot_ref, wrest_ref, b2_ref, bres2_ref,
               e8_ref, e8t_ref, y_ref, root_ref, res_ref):
    k = pl.program_id(0)
    rb = lax.broadcasted_iota(jnp.int32, (128, 128), 0) // C
    cb = lax.broadcasted_iota(jnp.int32, (128, 128), 1) // C
    mask = (rb == cb).astype(jnp.float32)
    e8t = e8t_ref[...]
    xg8 = xg8_ref[...]

    def blkdiag(w16):
        return pltpu.repeat(pltpu.repeat(w16, 8, 0), 8, 1) * mask

    y_ref[...] = _mm(xg8, blkdiag(wsp_ref[0]))

    @pl.when(k == NK - 1)
    def _():
        b8 = _mm(b2_ref[...], e8t)
        bres8 = _mm(bres2_ref[...], e8t)
        root_ref[...] = _mm(xg8, blkdiag(wroot_ref[...])) + b8
        rr = _mm(xg8, blkdiag(wrest_ref[...])) + bres8
        res_ref[...] = jnp.where(rr > 0, rr, jnp.exp(rr) - 1.0)


_prep = pl.pallas_call(
    _prep_body,
    grid=(NK,),
    in_specs=[
        pl.BlockSpec((_YR, 128), lambda k: (0, 0)),
        pl.BlockSpec((1, C, C), lambda k: (k, 0, 0)),
        pl.BlockSpec((C, C), lambda k: (0, 0)),
        pl.BlockSpec((C, C), lambda k: (0, 0)),
        pl.BlockSpec((1, C), lambda k: (0, 0)),
        pl.BlockSpec((1, C), lambda k: (0, 0)),
        pl.BlockSpec((128, C), lambda k: (0, 0)),
        pl.BlockSpec((C, 128), lambda k: (0, 0)),
    ],
    out_specs=[
        pl.BlockSpec((_YR, 128), lambda k: (k, 0)),
        pl.BlockSpec((_YR, 128), lambda k: (0, 0)),
        pl.BlockSpec((_YR, 128), lambda k: (0, 0)),
    ],
    out_shape=(
        jax.ShapeDtypeStruct((NK * _YR, 128), jnp.float32),
        jax.ShapeDtypeStruct((_YR, 128), jnp.float32),
        jax.ShapeDtypeStruct((_YR, 128), jnp.float32),
    ),
)


def _lane_bcast(vec, lane):
    return lax.gather(
        vec, jnp.full((16, 1), lane, jnp.int32),
        lax.GatherDimensionNumbers(offset_dims=(), collapsed_slice_dims=(0,),
                                   start_index_map=(0,)),
        (1,), mode=lax.GatherScatterMode.PROMISE_IN_BOUNDS)


def _edges_body(y_hbm, ea0_hbm, ea1_hbm, src_hbm, dst_hbm, root_hbm,
                res_hbm, out_hbm,
                ia0, ia1, ia2, ia3, ib0, ib1, ib2, ib3, dst_v,
                bas0, bas1, bas2, bas3, ea0_v, ea1_v, src_v,
                ra0, ra1, ra2, ra3, rb0, rb1, rb2, rb3,
                m_v, agg_sp, cnt_sp, sem_a, sem_b):
    c = lax.axis_index("c")
    s = lax.axis_index("s")
    e0 = s * EPT
    rpt = GPS * V // 16

    idx_a = (ia0, ia1, ia2, ia3)
    idx_b = (ib0, ib1, ib2, ib3)
    rows_a = (ra0, ra1, ra2, ra3)
    rows_b = (rb0, rb1, rb2, rb3)

    @plsc.parallel_loop(0, rpt, step=1, unroll=4)
    def _fill(j):
        m_v[j] = jnp.zeros((C,), jnp.float32)
        ra0[j] = jnp.ones((C,), jnp.float32)
        ra0[j + rpt] = jnp.ones((C,), jnp.float32)

    pltpu.sync_copy(m_v.at[pl.ds(0, rpt)], agg_sp.at[pl.ds(s * rpt, rpt)])
    pltpu.sync_copy(m_v.at[pl.ds(0, V // 16)],
                    cnt_sp.at[pl.ds(s * (V // 16), V // 16)])

    pltpu.sync_copy(ea0_hbm.at[pl.ds(e0, EPT)], ea0_v)
    pltpu.sync_copy(ea1_hbm.at[pl.ds(e0, EPT)], ea1_v)
    pltpu.sync_copy(src_hbm.at[pl.ds(e0, EPT)], src_v)
    pltpu.sync_copy(dst_hbm.at[pl.ds(e0, EPT)], dst_v)

    off0 = c * (GPS * V)

    @plsc.parallel_loop(0, EPT // 16, step=1, unroll=2)
    def _setup(j):
        sl = pl.ds(j * 16, 16)
        p0 = ea0_v[sl] * (K - 1.0)
        p1 = ea1_v[sl] * (K - 1.0)
        i0 = jnp.minimum(p0.astype(jnp.int32), K - 2)
        i1 = jnp.minimum(p1.astype(jnp.int32), K - 2)
        f0 = p0 - i0.astype(jnp.float32)
        f1 = p1 - i1.astype(jnp.float32)
        g0 = 1.0 - f0
        g1 = 1.0 - f1
        bas0[sl] = g0 * g1
        bas1[sl] = f0 * g1
        bas2[sl] = g0 * f1
        bas3[sl] = f0 * f1
        wib = src_v[sl] + (i0 + i1 * K) * NODES + off0
        ia0[sl] = wib
        ia1[sl] = wib + NODES
        ia2[sl] = wib + K * NODES
        ia3[sl] = wib + (K + 1) * NODES

    def _addv(dref, sref, nchunks, val):
        @plsc.parallel_loop(0, nchunks, step=1, unroll=2)
        def f(j):
            dref[pl.ds(j * 16, 16)] = sref[pl.ds(j * 16, 16)] + val

    plsc.subcore_barrier()

    pltpu.sync_copy(ra0, cnt_sp.at[dst_v], add=True)

    bufs = ((idx_a, rows_a, sem_a), (idx_b, rows_b, sem_b))
    descs = [pltpu.async_copy(y_hbm.at[ix], r, sem_a)
             for ix, r in zip(idx_a, rows_a)]
    for g in range(GPS):
        cur_i, cur_r, _ = bufs[g % 2]
        nxt_i, nxt_r, nxt_s = bufs[(g + 1) % 2]
        if g < GPS - 1:
            for corner in range(4):
                _addv(nxt_i[corner], cur_i[corner], EPT // 16, V)
        cur_descs = descs
        if g < GPS - 1:
            descs = [pltpu.async_copy(y_hbm.at[ix], r, nxt_s)
                     for ix, r in zip(nxt_i, nxt_r)]
        for d in cur_descs:
            d.wait()

        c0, c1, c2, c3 = cur_r

        @plsc.parallel_loop(0, EPT // 16, step=1, unroll=1)
        def _body(ch):
            base = ch * 16
            sl = pl.ds(base, 16)
            b0 = bas0[sl]
            b1 = bas1[sl]
            b2v = bas2[sl]
            b3 = bas3[sl]
            for u in range(16):
                ee = base + u
                m = (c0[ee] * _lane_bcast(b0, u)
                     + c1[ee] * _lane_bcast(b1, u)
                     + c2[ee] * _lane_bcast(b2v, u)
                     + c3[ee] * _lane_bcast(b3, u))
                m_v[ee] = m

        pltpu.sync_copy(m_v, agg_sp.at[dst_v], add=True)
        if g < GPS - 1:
            _addv(dst_v, dst_v, EPT // 16, V)

    plsc.subcore_barrier()

    row0 = s * rpt
    gbase = c * (GPS * V) + row0
    v_off = lax.rem(s, 2) * rpt
    pltpu.sync_copy(agg_sp.at[pl.ds(row0, rpt)], m_v.at[pl.ds(0, rpt)])
    pltpu.sync_copy(cnt_sp.at[pl.ds(v_off, rpt)], ra2.at[pl.ds(0, rpt)])
    pltpu.sync_copy(root_hbm.at[pl.ds(gbase, rpt)], ra0.at[pl.ds(0, rpt)])
    pltpu.sync_copy(res_hbm.at[pl.ds(gbase, rpt)], ra1.at[pl.ds(0, rpt)])

    def _fin(j, carry):
        cntv = jnp.maximum(ra2[j], 1.0)
        xo = m_v[j] / cntv + ra0[j]
        xo = jnp.where(xo > 0, xo, jnp.exp(xo) - 1.0)
        xo = xo + ra1[j]
        xo = jnp.where(xo > 0, xo, jnp.exp(xo) - 1.0)
        m_v[j + rpt] = xo
        return carry
    lax.fori_loop(0, rpt, _fin, 0)
    pltpu.sync_copy(m_v.at[pl.ds(rpt, rpt)], out_hbm.at[pl.ds(gbase, rpt)])


@functools.cache
def _get_edges():
    mesh = plsc.VectorSubcoreMesh(core_axis_name="c", subcore_axis_name="s",
                                  num_cores=2, num_subcores=16)
    idx_t = pltpu.VMEM((EPT,), jnp.int32)
    vec_t = pltpu.VMEM((EPT,), jnp.float32)
    row_t = pltpu.VMEM((EPT, C), jnp.float32)
    return pl.kernel(
        _edges_body,
        mesh=mesh,
        compiler_params=pltpu.CompilerParams(use_tc_tiling_on_sc=False),
        out_type=jax.ShapeDtypeStruct((NODES, C), jnp.float32),
        scratch_types=[
            idx_t, idx_t, idx_t, idx_t,
            idx_t, idx_t, idx_t, idx_t,
            idx_t,
            vec_t, vec_t, vec_t, vec_t,
            vec_t, vec_t,
            idx_t,
            row_t, row_t, row_t, row_t,
            row_t, row_t, row_t, row_t,
            row_t,
            pltpu.VMEM_SHARED((GPS * V, C), jnp.float32),
            pltpu.VMEM_SHARED((V, C), jnp.float32),
            pltpu.SemaphoreType.DMA,
            pltpu.SemaphoreType.DMA,
        ],
    )


def kernel(x, edge_index, edge_attr, Wspline, Wroot, b, Wres, bres):
    n, v, cc, t = x.shape
    xg = x.transpose(3, 0, 1, 2).reshape(NODES, C)

    ea0 = edge_attr[:NEB, 0]
    ea1 = edge_attr[:NEB, 1]
    src = edge_index[0, :NEB]
    dst = edge_index[1, :NEB]

    e8 = jnp.tile(jnp.eye(C, dtype=jnp.float32), (8, 1))
    b2 = b.reshape(1, C)
    bres2 = bres.reshape(1, C)
    xg128 = xg.reshape(_YR, 128)

    y, root128, res128 = _prep(xg128, Wspline, Wroot, Wres.T, b2, bres2,
                               e8, e8.T)
    yflat = y.reshape(NK * NODES, C)
    root = root128.reshape(NODES, C)
    res = res128.reshape(NODES, C)

    out_node = _get_edges()(yflat, ea0, ea1, src, dst, root, res)

    return out_node.reshape(t, n, v, C).transpose(1, 2, 3, 0)
